# baseline (speedup 1.0000x reference)
import jax
import jax.numpy as jnp
from jax.experimental import pallas as pl
from jax.experimental.pallas import tpu as pltpu

N_GROUPS_ = 8
GROUP_SIZE_ = 8
N_EXPERTS_ = 64
TOPK_GROUPS_ = 4
TOPK_ = 2
NEG_INF_ = float("-inf")
NBLK_ = 8


def _routing_epilogue(st, w_out_ref, i_out_ref):
    bt = st.shape[1]

    row_max = jnp.max(st, axis=0, keepdims=True)
    e = jnp.exp(st - row_max)
    p = e / jnp.sum(e, axis=0, keepdims=True)

    gms = [
        jnp.max(p[g * GROUP_SIZE_ : (g + 1) * GROUP_SIZE_], axis=0, keepdims=True)
        for g in range(N_GROUPS_)
    ]

    sels = []
    for g in range(N_GROUPS_):
        cnt = None
        for h in range(N_GROUPS_):
            if h == g:
                continue
            if h < g:
                beats = gms[h] >= gms[g]
            else:
                beats = gms[h] > gms[g]
            b = beats.astype(jnp.int32)
            cnt = b if cnt is None else cnt + b
        sels.append(cnt < TOPK_GROUPS_)

    masked = jnp.concatenate(
        [
            jnp.where(sels[g], p[g * GROUP_SIZE_ : (g + 1) * GROUP_SIZE_], NEG_INF_)
            for g in range(N_GROUPS_)
        ],
        axis=0,
    )

    expert_id = jax.lax.broadcasted_iota(jnp.int32, (N_EXPERTS_, bt), 0)

    ws = []
    idxs = []
    for _ in range(TOPK_):
        vmax = jnp.max(masked, axis=0, keepdims=True)
        cand = jnp.where(masked == vmax, expert_id, N_EXPERTS_)
        win = jnp.min(cand, axis=0, keepdims=True)
        ws.append(vmax)
        idxs.append(win)
        masked = jnp.where(expert_id == win, NEG_INF_, masked)

    w_out_ref[...] = jnp.concatenate(ws, axis=0)
    i_out_ref[...] = jnp.concatenate(idxs, axis=0)


def _gate_kernel(xa_ref, xb_ref, w_ref, w_out_ref, i_out_ref, s_ref):
    i = pl.program_id(0)

    @pl.when(i < NBLK_)
    def _matmul():
        for h, xr in enumerate((xa_ref, xb_ref)):
            sub = xr.shape[0]
            s_ref[i % 2, :, pl.ds(h * sub, sub)] = jax.lax.dot_general(
                w_ref[...],
                xr[...],
                (((1,), (1,)), ((), ())),
                preferred_element_type=jnp.float32,
            )

    @pl.when(i > 0)
    def _epilogue():
        _routing_epilogue(s_ref[(i + 1) % 2], w_out_ref, i_out_ref)


@jax.jit
def kernel(x, W):
    T, D = x.shape
    bt = T // NBLK_
    grid = (NBLK_ + 1,)
    weights_t, indices_t = pl.pallas_call(
        _gate_kernel,
        grid=grid,
        in_specs=[
            pl.BlockSpec((bt // 2, D), lambda i: (2 * jnp.minimum(i, NBLK_ - 1), 0)),
            pl.BlockSpec(
                (bt // 2, D), lambda i: (2 * jnp.minimum(i, NBLK_ - 1) + 1, 0)
            ),
            pl.BlockSpec((N_EXPERTS_, D), lambda i: (0, 0)),
        ],
        out_specs=[
            pl.BlockSpec((TOPK_, bt), lambda i: (0, jnp.maximum(i - 1, 0))),
            pl.BlockSpec((TOPK_, bt), lambda i: (0, jnp.maximum(i - 1, 0))),
        ],
        out_shape=[
            jax.ShapeDtypeStruct((TOPK_, T), jnp.float32),
            jax.ShapeDtypeStruct((TOPK_, T), jnp.int32),
        ],
        scratch_shapes=[pltpu.VMEM((2, N_EXPERTS_, bt), jnp.float32)],
        compiler_params=pltpu.CompilerParams(
            dimension_semantics=("arbitrary",),
        ),
    )(x, x, W)
    return weights_t.T.astype(x.dtype), indices_t.T


# speedup vs baseline: 1.0195x; 1.0195x over previous
import jax
import jax.numpy as jnp
from jax.experimental import pallas as pl
from jax.experimental.pallas import tpu as pltpu

N_GROUPS_ = 8
GROUP_SIZE_ = 8
N_EXPERTS_ = 64
TOPK_GROUPS_ = 4
TOPK_ = 2
NEG_INF_ = float("-inf")


def _gate_kernel(xa_ref, xb_ref, w_ref, w_out_ref, i_out_ref):
    sub = xa_ref.shape[0]
    for h, xr in enumerate((xa_ref, xb_ref)):
        _gate_subtile(
            xr[...],
            w_ref[...],
            w_out_ref.at[:, pl.ds(h * sub, sub)],
            i_out_ref.at[:, pl.ds(h * sub, sub)],
        )


def _gate_subtile(x, w, w_out_ref, i_out_ref):
    st = jax.lax.dot_general(
        w,
        x,
        (((1,), (1,)), ((), ())),
        preferred_element_type=jnp.float32,
    )
    bt = st.shape[1]

    row_max = jnp.max(st, axis=0, keepdims=True)
    e = jnp.exp(st - row_max)
    p = e / jnp.sum(e, axis=0, keepdims=True)

    gms = [
        jnp.max(p[g * GROUP_SIZE_ : (g + 1) * GROUP_SIZE_], axis=0, keepdims=True)
        for g in range(N_GROUPS_)
    ]

    sels = []
    for g in range(N_GROUPS_):
        cnt = None
        for h in range(N_GROUPS_):
            if h == g:
                continue
            if h < g:
                beats = gms[h] >= gms[g]
            else:
                beats = gms[h] > gms[g]
            b = beats.astype(jnp.int32)
            cnt = b if cnt is None else cnt + b
        sels.append(cnt < TOPK_GROUPS_)

    masked = jnp.concatenate(
        [
            jnp.where(
                sels[g], p[g * GROUP_SIZE_ : (g + 1) * GROUP_SIZE_], NEG_INF_
            )
            for g in range(N_GROUPS_)
        ],
        axis=0,
    )

    expert_id = jax.lax.broadcasted_iota(jnp.int32, (N_EXPERTS_, bt), 0)

    ws = []
    idxs = []
    for _ in range(TOPK_):
        vmax = jnp.max(masked, axis=0, keepdims=True)
        cand = jnp.where(masked == vmax, expert_id, N_EXPERTS_)
        win = jnp.min(cand, axis=0, keepdims=True)
        ws.append(vmax)
        idxs.append(win)
        masked = jnp.where(expert_id == win, NEG_INF_, masked)

    w_out_ref[...] = jnp.concatenate(ws, axis=0)
    i_out_ref[...] = jnp.concatenate(idxs, axis=0)


@jax.jit
def kernel(x, W):
    T, D = x.shape
    bt = 1024
    grid = (T // bt,)
    weights_t, indices_t = pl.pallas_call(
        _gate_kernel,
        grid=grid,
        in_specs=[
            pl.BlockSpec((bt // 2, D), lambda i: (2 * i, 0)),
            pl.BlockSpec((bt // 2, D), lambda i: (2 * i + 1, 0)),
            pl.BlockSpec((N_EXPERTS_, D), lambda i: (0, 0)),
        ],
        out_specs=[
            pl.BlockSpec((TOPK_, bt), lambda i: (0, i)),
            pl.BlockSpec((TOPK_, bt), lambda i: (0, i)),
        ],
        out_shape=[
            jax.ShapeDtypeStruct((TOPK_, T), jnp.float32),
            jax.ShapeDtypeStruct((TOPK_, T), jnp.int32),
        ],
        compiler_params=pltpu.CompilerParams(
            dimension_semantics=("arbitrary",),
        ),
    )(x, x, W)
    return weights_t.T.astype(x.dtype), indices_t.T
